# baseline (speedup 1.0000x reference)
import functools

import jax
import jax.numpy as jnp
from jax import lax
from jax.experimental import pallas as pl
from jax.experimental.pallas import tpu as pltpu
from jax.experimental.pallas import tpu_sc as plsc

B = 16384
EMB = 32
NC = 2
NS = 16
NW = NC * NS
CHUNK = 128
ROWS = B // CHUNK
RPW = ROWS // NW


def _gather_body(uidx_hbm, midx_hbm, utab_hbm, mtab_hbm, ue_out, me_out,
                 uidx_v, midx_v, urows_v, mrows_v, sem):
    wid = lax.axis_index("s") * NC + lax.axis_index("c")
    r0 = wid * RPW
    pltpu.sync_copy(uidx_hbm.at[pl.ds(r0, RPW)], uidx_v)
    pltpu.sync_copy(midx_hbm.at[pl.ds(r0, RPW)], midx_v)
    copies = []
    for j in range(RPW):
        copies.append(pltpu.async_copy(utab_hbm.at[uidx_v.at[j]],
                                       urows_v.at[j], sem))
        copies.append(pltpu.async_copy(mtab_hbm.at[midx_v.at[j]],
                                       mrows_v.at[j], sem))
    for c in copies:
        c.wait()
    pltpu.sync_copy(urows_v, ue_out.at[pl.ds(r0, RPW)])
    pltpu.sync_copy(mrows_v, me_out.at[pl.ds(r0, RPW)])


@functools.partial(
    pl.kernel,
    mesh=plsc.VectorSubcoreMesh(core_axis_name="c", subcore_axis_name="s",
                                num_cores=NC),
    compiler_params=pltpu.CompilerParams(use_tc_tiling_on_sc=False),
    out_type=[
        jax.ShapeDtypeStruct((ROWS, CHUNK, EMB), jnp.float32),
        jax.ShapeDtypeStruct((ROWS, CHUNK, EMB), jnp.float32),
    ],
    scratch_types=[
        pltpu.VMEM((RPW, CHUNK), jnp.int32),
        pltpu.VMEM((RPW, CHUNK), jnp.int32),
        pltpu.VMEM((RPW, CHUNK, EMB), jnp.float32),
        pltpu.VMEM((RPW, CHUNK, EMB), jnp.float32),
        pltpu.SemaphoreType.DMA,
    ],
)
def _gather(*args):
    _gather_body(*args)


def _mlp_body(ue_ref, me_ref, w1a_ref, w1b_ref, b1_ref, w2_ref, b2_ref,
              w3_ref, b3_ref, out_ref):
    x = jnp.dot(ue_ref[...], w1a_ref[...], preferred_element_type=jnp.float32)
    x = x + jnp.dot(me_ref[...], w1b_ref[...],
                    preferred_element_type=jnp.float32)
    x = jnp.maximum(x + b1_ref[...], 0.0)
    x = jnp.maximum(
        jnp.dot(x, w2_ref[...], preferred_element_type=jnp.float32)
        + b2_ref[...], 0.0)
    out_ref[...] = (jnp.dot(x, w3_ref[...], preferred_element_type=jnp.float32)
                    + b3_ref[...])


def _mlp(ue, me, W1a, W1b, b1, W2, b2, W3, b3):
    BB = 2048
    grid = (B // BB,)
    full = lambda shape: pl.BlockSpec(shape, lambda i: (0, 0))
    return pl.pallas_call(
        _mlp_body,
        grid=grid,
        in_specs=[
            pl.BlockSpec((BB, EMB), lambda i: (i, 0)),
            pl.BlockSpec((BB, EMB), lambda i: (i, 0)),
            full((EMB, 64)),
            full((EMB, 64)),
            full((1, 64)),
            full((64, 32)),
            full((1, 32)),
            full((32, 1)),
            full((1, 1)),
        ],
        out_specs=pl.BlockSpec((BB, 1), lambda i: (i, 0)),
        out_shape=jax.ShapeDtypeStruct((B, 1), jnp.float32),
    )(ue, me, W1a, W1b, b1, W2, b2, W3, b3)


def kernel(user, movie, user_table, movie_table, W1, b1, W2, b2, W3, b3):
    u2 = user.astype(jnp.int32).reshape(ROWS, CHUNK)
    m2 = movie.astype(jnp.int32).reshape(ROWS, CHUNK)
    ue, me = _gather(u2, m2, user_table, movie_table)
    ue = ue.reshape(B, EMB)
    me = me.reshape(B, EMB)
    return _mlp(ue, me, W1[:EMB], W1[EMB:], b1.reshape(1, 64),
                W2, b2.reshape(1, 32), W3, b3.reshape(1, 1))
